# Initial kernel scaffold; baseline (speedup 1.0000x reference)
import jax
import jax.numpy as jnp
from jax.experimental import pallas as pl


def kernel(center_words, context_words, neg_samples, in_embed, out_embed):
    raise NotImplementedError("write your pallas kernel here")


# speedup vs baseline: 4.7940x; 4.7940x over previous
import functools

import jax
import jax.numpy as jnp
from jax import lax
from jax.experimental import pallas as pl
from jax.experimental.pallas import tpu as pltpu
from jax.experimental.pallas import tpu_sc as plsc

DIM = 64
B = 16384
K = 20

NC, NS, LANES = 2, 16, 16
NW = NC * NS
BW = B // NW
CB = 32
NCHUNK = BW // CB
CK = CB * K
NEG_SEG = 128
NSEG = CK // NEG_SEG


def _sc_scores(cen_idx, ctx_idx, neg_idx, in_embed, out_embed):
    mesh = plsc.VectorSubcoreMesh(
        core_axis_name="c", subcore_axis_name="s", num_cores=NC, num_subcores=NS
    )

    @functools.partial(
        pl.kernel,
        mesh=mesh,
        out_type=(
            jax.ShapeDtypeStruct((B,), jnp.float32),
            jax.ShapeDtypeStruct((B * K,), jnp.float32),
        ),
        compiler_params=pltpu.CompilerParams(
            use_tc_tiling_on_sc=False, needs_layout_passes=False
        ),
        scratch_types=[
            pltpu.VMEM((CB,), jnp.int32),
            pltpu.VMEM((CB,), jnp.int32),
            pltpu.VMEM((CB,), jnp.int32),
            pltpu.VMEM((CB,), jnp.int32),
            pltpu.VMEM((CK,), jnp.int32),
            pltpu.VMEM((CK,), jnp.int32),
            pltpu.VMEM((CB, DIM), jnp.float32),
            pltpu.VMEM((CB, DIM), jnp.float32),
            pltpu.VMEM((CB, DIM), jnp.float32),
            pltpu.VMEM((CB, DIM), jnp.float32),
            pltpu.VMEM((CK, DIM), jnp.float32),
            pltpu.VMEM((CK, DIM), jnp.float32),
            pltpu.VMEM((CB,), jnp.float32),
            pltpu.VMEM((CK,), jnp.float32),
            pltpu.SemaphoreType.DMA,
            pltpu.SemaphoreType.DMA,
        ],
    )
    def k(cen_hbm, ctx_hbm, neg_hbm, ine_hbm, oute_hbm, pos_out, neg_out,
          ci0, ci1, xi0, xi1, ni0, ni1, cr0, cr1, xr0, xr1, nr0, nr1,
          pv, nv, sem0, sem1):
        wid = lax.axis_index("s") * NC + lax.axis_index("c")
        ci = (ci0, ci1)
        xi = (xi0, xi1)
        ni = (ni0, ni1)
        cr = (cr0, cr1)
        xr = (xr0, xr1)
        nr = (nr0, nr1)
        sem = (sem0, sem1)
        lane = lax.iota(jnp.int32, LANES)
        m15 = lane == (LANES - 1)

        def fire(c, slot):
            base = wid * BW + c * CB
            pltpu.sync_copy(cen_hbm.at[pl.ds(base, CB)], ci[slot])
            pltpu.sync_copy(ctx_hbm.at[pl.ds(base, CB)], xi[slot])
            pltpu.sync_copy(neg_hbm.at[pl.ds(base * K, CK)], ni[slot])
            hs = [
                pltpu.async_copy(ine_hbm.at[ci[slot]], cr[slot], sem[slot]),
                pltpu.async_copy(oute_hbm.at[xi[slot]], xr[slot], sem[slot]),
            ]
            for s in range(NSEG):
                sl = pl.ds(s * NEG_SEG, NEG_SEG)
                hs.append(
                    pltpu.async_copy(
                        oute_hbm.at[ni[slot].at[sl]], nr[slot].at[sl], sem[slot]
                    )
                )
            return hs

        def dot(row_ref, cvs):
            m = row_ref[pl.ds(0, LANES)] * cvs[0]
            m = m + row_ref[pl.ds(LANES, LANES)] * cvs[1]
            m = m + row_ref[pl.ds(2 * LANES, LANES)] * cvs[2]
            m = m + row_ref[pl.ds(3 * LANES, LANES)] * cvs[3]
            return plsc.cumsum(m)

        def compute(slot):
            def b_body(b, carry):
                crow = cr[slot].at[b]
                cvs = [crow[pl.ds(i * LANES, LANES)] for i in range(4)]
                s = dot(xr[slot].at[b], cvs)
                plsc.store_scatter(
                    pv, [jnp.full((LANES,), b, jnp.int32)], s, mask=m15
                )
                p0 = b * K
                for j in range(K):
                    s = dot(nr[slot].at[p0 + j], cvs)
                    plsc.store_scatter(
                        nv, [jnp.full((LANES,), p0 + j, jnp.int32)], s, mask=m15
                    )
                return carry

            lax.fori_loop(0, CB, b_body, 0)

        handles = [None, None]
        handles[0] = fire(0, 0)
        for c in range(NCHUNK):
            slot = c & 1
            if c + 1 < NCHUNK:
                handles[1 - slot] = fire(c + 1, 1 - slot)
            for h in handles[slot]:
                h.wait()
            compute(slot)
            base = wid * BW + c * CB
            pltpu.sync_copy(pv, pos_out.at[pl.ds(base, CB)])
            pltpu.sync_copy(nv, neg_out.at[pl.ds(base * K, CK)])

    return k(cen_idx, ctx_idx, neg_idx, in_embed, out_embed)


def kernel(center_words, context_words, neg_samples, in_embed, out_embed):
    cen_idx = center_words.astype(jnp.int32)
    ctx_idx = context_words.astype(jnp.int32)
    neg_idx = neg_samples.astype(jnp.int32).reshape(-1)
    pos, neg = _sc_scores(cen_idx, ctx_idx, neg_idx, in_embed, out_embed)
    return (pos.reshape(B, 1), neg.reshape(B, K))


# speedup vs baseline: 8.5395x; 1.7813x over previous
import functools

import jax
import jax.numpy as jnp
from jax import lax
from jax.experimental import pallas as pl
from jax.experimental.pallas import tpu as pltpu
from jax.experimental.pallas import tpu_sc as plsc

DIM = 64
B = 16384
K = 20
VOC = 1000000
VPK = ((VOC - 1) >> 8) * 128 + ((VOC - 1) & 127) + 1
RVB = 16384

NC, NS, LANES = 2, 16, 16
NW = NC * NS
BW = B // NW
CB = 16
NCHUNK = BW // CB
CK = CB * K
NEG_SEGS = ((0, 128), (128, 128), (256, 64))


@functools.partial(
    pl.pallas_call,
    grid=((VOC + RVB - 1) // RVB,),
    in_specs=[pl.BlockSpec((DIM, RVB), lambda i: (0, i))],
    out_specs=pl.BlockSpec((RVB // 2, 128), lambda i: (i, 0)),
    out_shape=jax.ShapeDtypeStruct((VPK, 128), jnp.float32),
)
def _repack(x_ref, o_ref):
    eye = jnp.eye(DIM, dtype=jnp.float32)
    xt = jax.lax.dot_general(
        x_ref[...], eye, (((0,), (0,)), ((), ())),
        preferred_element_type=jnp.float32,
    )
    parts = []
    for g in range(RVB // 256):
        lo = xt[g * 256: g * 256 + 128]
        hi = xt[g * 256 + 128: (g + 1) * 256]
        parts.append(jnp.concatenate([lo, hi], axis=1))
    o_ref[...] = jnp.concatenate(parts, axis=0)


IX_CEN = 0
IX_CTX = CB
IX_CPK = 2 * CB
IX_XPK = 3 * CB
IX_NEG = 4 * CB
IX_NPK = 4 * CB + CK
IXLEN = 4 * CB + 2 * CK


def _sc_scores(idx_pack, in_pack, out_pack):
    mesh = plsc.VectorSubcoreMesh(
        core_axis_name="c", subcore_axis_name="s", num_cores=NC, num_subcores=NS
    )

    @functools.partial(
        pl.kernel,
        mesh=mesh,
        out_type=(
            jax.ShapeDtypeStruct((B,), jnp.float32),
            jax.ShapeDtypeStruct((B * K,), jnp.float32),
        ),
        compiler_params=pltpu.CompilerParams(
            use_tc_tiling_on_sc=False, needs_layout_passes=False
        ),
        scratch_types=[
            pltpu.VMEM((2, IXLEN), jnp.int32),
            pltpu.VMEM((2, CB, 2 * DIM), jnp.float32),
            pltpu.VMEM((2, CB, 2 * DIM), jnp.float32),
            pltpu.VMEM((2, CK, 2 * DIM), jnp.float32),
            pltpu.VMEM((2, CB), jnp.float32),
            pltpu.VMEM((2, CK), jnp.float32),
            pltpu.SemaphoreType.DMA,
            pltpu.SemaphoreType.DMA,
            pltpu.SemaphoreType.DMA,
            pltpu.SemaphoreType.DMA,
        ],
    )
    def k(ixp_hbm, inp_hbm, outp_hbm, pos_out, neg_out,
          ixv, crp, xrp, nrp, pv, nv, sem0, sem1, so0, so1):
        wid = lax.axis_index("s") * NC + lax.axis_index("c")
        sem = (sem0, sem1)
        sout = (so0, so1)
        lane = lax.iota(jnp.int32, LANES)
        m0 = lane == 0

        def stage(c, slot):
            g = wid * NCHUNK + c
            pltpu.sync_copy(ixp_hbm.at[pl.ds(g * IXLEN, IXLEN)], ixv.at[slot])

        def gather_descs(slot):
            ds = [
                (inp_hbm.at[ixv.at[slot].at[pl.ds(IX_CPK, CB)]], crp.at[slot]),
                (outp_hbm.at[ixv.at[slot].at[pl.ds(IX_XPK, CB)]], xrp.at[slot]),
            ]
            for off, ln in NEG_SEGS:
                ds.append((outp_hbm.at[ixv.at[slot].at[pl.ds(IX_NPK + off, ln)]],
                           nrp.at[slot].at[pl.ds(off, ln)]))
            return ds

        def fire(c, slot):
            stage(c, slot)
            for src, dst in gather_descs(slot):
                pltpu.async_copy(src, dst, sem[slot])

        def drain(slot):
            for src, dst in gather_descs(slot):
                pltpu.make_async_copy(src, dst, sem[slot]).wait()

        def butterfly(m):
            for sh in (8, 4, 2, 1):
                m = m + jnp.take(m, lane ^ sh)
            return m

        def dotp(row_ref, off, cvs):
            m = row_ref[pl.ds(off, LANES)] * cvs[0]
            m = m + row_ref[pl.ds(off + LANES, LANES)] * cvs[1]
            m = m + row_ref[pl.ds(off + 2 * LANES, LANES)] * cvs[2]
            m = m + row_ref[pl.ds(off + 3 * LANES, LANES)] * cvs[3]
            return butterfly(m)

        def out_descs(c, slot):
            base = wid * BW + c * CB
            return [
                (pv.at[slot], pos_out.at[pl.ds(base, CB)]),
                (nv.at[slot], neg_out.at[pl.ds(base * K, CK)]),
            ]

        def compute(c, slot):
            @pl.when(c >= 2)
            def _():
                for src, dst in out_descs(c - 2, slot):
                    pltpu.make_async_copy(src, dst, sout[slot]).wait()

            civ = ixv[slot, pl.ds(IX_CEN, LANES)]
            xiv = ixv[slot, pl.ds(IX_CTX, LANES)]

            def b_body(b, carry):
                coff = ((jnp.take(civ, jnp.full((LANES,), b, jnp.int32))[0]
                         >> 7) & 1) * DIM
                xoff = ((jnp.take(xiv, jnp.full((LANES,), b, jnp.int32))[0]
                         >> 7) & 1) * DIM
                crow = crp.at[slot].at[b]
                cvs = [crow[pl.ds(coff + i * LANES, LANES)] for i in range(4)]
                s = dotp(xrp.at[slot].at[b], xoff, cvs)
                plsc.store_scatter(
                    pv.at[slot], [jnp.full((LANES,), b, jnp.int32)], s, mask=m0
                )
                p0 = b * K
                nv0 = ixv[slot, pl.ds(IX_NEG + p0, LANES)]
                nv1 = ixv[slot, pl.ds(IX_NEG + p0 + LANES, LANES)]
                for j in range(K):
                    vj = nv0[j] if j < LANES else nv1[j - LANES]
                    noff = ((vj >> 7) & 1) * DIM
                    s = dotp(nrp.at[slot].at[p0 + j], noff, cvs)
                    plsc.store_scatter(
                        nv.at[slot], [jnp.full((LANES,), p0 + j, jnp.int32)],
                        s, mask=m0
                    )
                return carry

            lax.fori_loop(0, CB, b_body, 0)
            for src, dst in out_descs(c, slot):
                pltpu.async_copy(src, dst, sout[slot])

        fire(0, 0)

        def step_body(step, carry):
            c0 = 2 * step
            fire(c0 + 1, 1)
            drain(0)
            compute(c0, 0)

            @pl.when(step < NCHUNK // 2 - 1)
            def _():
                fire(c0 + 2, 0)

            drain(1)
            compute(c0 + 1, 1)
            return carry

        lax.fori_loop(0, NCHUNK // 2, step_body, 0)
        for src, dst in out_descs(NCHUNK - 2, 0):
            pltpu.make_async_copy(src, dst, sout[0]).wait()
        for src, dst in out_descs(NCHUNK - 1, 1):
            pltpu.make_async_copy(src, dst, sout[1]).wait()

    return k(idx_pack, in_pack, out_pack)


def kernel(center_words, context_words, neg_samples, in_embed, out_embed):
    one_i = (center_words[0] >= 0).astype(jnp.int32) | jnp.int32(1)
    one_f = one_i.astype(jnp.float32)
    cen_idx = center_words.astype(jnp.int32)
    ctx_idx = context_words.astype(jnp.int32)
    neg_idx = neg_samples.astype(jnp.int32).reshape(-1) * one_i
    cen_pk = ((cen_idx >> 8) << 7) | (cen_idx & 127)
    ctx_pk = ((ctx_idx >> 8) << 7) | (ctx_idx & 127)
    neg_pk = ((neg_idx >> 8) << 7) | (neg_idx & 127)
    nch = B // CB
    idx_pack = jnp.concatenate(
        [cen_idx.reshape(nch, CB), ctx_idx.reshape(nch, CB),
         cen_pk.reshape(nch, CB), ctx_pk.reshape(nch, CB),
         neg_idx.reshape(nch, CK), neg_pk.reshape(nch, CK)],
        axis=1,
    ).reshape(-1) * one_i
    in_pack = _repack(in_embed.T)
    out_pack = _repack(out_embed.T)
    pos, neg = _sc_scores(idx_pack, in_pack, out_pack)
    return (pos.reshape(B, 1) * one_f, neg.reshape(B, K) * one_f)
